# baseline (speedup 1.0000x reference)
import functools

import jax
import jax.numpy as jnp
from jax import lax
from jax.experimental import pallas as pl
from jax.experimental.pallas import tpu as pltpu
from jax.experimental.pallas import tpu_sc as plsc


@functools.cache
def _build(batch: int, hist: int, vocab: int, d: int):
  info = plsc.get_sparse_core_info()
  nc, ns = info.num_cores, info.num_subcores
  nw = nc * ns
  bpw = batch // nw
  nbuf = 5
  steps = hist // nbuf
  assert batch == nw * bpw and hist == steps * nbuf

  mesh = plsc.VectorSubcoreMesh(core_axis_name="c", subcore_axis_name="s")

  def body(idx_hbm, table_hbm, out_hbm, idx_v, bufs, gsems, wsems):
    wid = lax.axis_index("s") * nc + lax.axis_index("c")
    b0 = wid * bpw

    pltpu.sync_copy(idx_hbm.at[:, pl.ds(b0, bpw)], idx_v)

    def gather(h, b):
      pltpu.async_copy(table_hbm.at[idx_v.at[h]], bufs[b], gsems[b])

    def gwait(b):
      pltpu.make_async_copy(
          table_hbm.at[idx_v.at[0]], bufs[b], gsems[b]).wait()

    def wwait(b):
      pltpu.make_async_copy(
          bufs[b], out_hbm.at[0, pl.ds(b0, bpw)], wsems[b]).wait()

    for b in range(nbuf):
      gather(b, b)

    def step(i, carry):
      h0 = nbuf * i
      for b in range(nbuf):
        gwait(b)
        pltpu.async_copy(bufs[b], out_hbm.at[h0 + b, pl.ds(b0, bpw)],
                         wsems[b])

        if b >= 1:
          @pl.when(i < steps - 1)
          def _(b=b):
            wwait(b - 1)
            gather(h0 + nbuf + b - 1, b - 1)

      @pl.when(i < steps - 1)
      def _():
        wwait(nbuf - 1)
        gather(h0 + 2 * nbuf - 1, nbuf - 1)

      return carry

    lax.fori_loop(0, steps, step, 0)

    for b in range(nbuf):
      wwait(b)

  return pl.kernel(
      body,
      out_type=jax.ShapeDtypeStruct((hist, batch, d), jnp.float32),
      mesh=mesh,
      scratch_types=[
          pltpu.VMEM((hist, bpw), jnp.int32),
          [pltpu.VMEM((bpw, d), jnp.float32) for _ in range(nbuf)],
          [pltpu.SemaphoreType.DMA for _ in range(nbuf)],
          [pltpu.SemaphoreType.DMA for _ in range(nbuf)],
      ],
  )


@jax.jit
def kernel(x, table):
  b, h = x.shape
  vocab, d = table.shape
  out_t = _build(b, h, vocab, d)(x.T.astype(jnp.int32), table)
  return out_t.transpose(1, 0, 2)
